# Initial kernel scaffold; baseline (speedup 1.0000x reference)
import functools

import jax
import jax.numpy as jnp
from jax import lax
from jax.experimental import pallas as pl
from jax.experimental.pallas import tpu as pltpu
from jax.experimental.pallas import tpu_sc as plsc

NM = 10000
ND = 10000
D = 256
H = 256
E = 160000
L_EDGES = 16384

NC = 2
NS = 16
LANES = 16

EPAD = 163840
EPT = EPAD // NS
NBLK = EPT // 128
ACC_ROWS = 10240
PAD_DST = 10016
ZROWS = 160
WROWS = 625

_MESH = plsc.VectorSubcoreMesh(core_axis_name="c", subcore_axis_name="s")


_RB = 2000


def _proj_body(x_ref, w_ref, b_ref, lo_ref, hi_ref):
    y = jnp.dot(x_ref[...], w_ref[...], preferred_element_type=jnp.float32)
    y = y + b_ref[...]
    lo_ref[...] = y[:, :128]
    hi_ref[...] = y[:, 128:]


def _proj(x, w, b):
    n = x.shape[0]
    grid = (n // _RB,)
    return pl.pallas_call(
        _proj_body,
        grid=grid,
        in_specs=[
            pl.BlockSpec((_RB, D), lambda i: (i, 0)),
            pl.BlockSpec((D, H), lambda i: (0, 0)),
            pl.BlockSpec((1, H), lambda i: (0, 0)),
        ],
        out_specs=[
            pl.BlockSpec((_RB, 128), lambda i: (i, 0)),
            pl.BlockSpec((_RB, 128), lambda i: (i, 0)),
        ],
        out_shape=[
            jax.ShapeDtypeStruct((n, 128), jnp.float32),
            jax.ShapeDtypeStruct((n, 128), jnp.float32),
        ],
    )(x, w, b.reshape(1, H))


def _comb_body(m_ref, hlo_ref, hhi_ref, wl_ref, wr_ref, b_ref, lo_ref, hi_ref,
               *, relu):
    mlo = m_ref[0]
    mhi = m_ref[1]
    y = jnp.dot(mlo, wl_ref[:128, :], preferred_element_type=jnp.float32)
    y += jnp.dot(mhi, wl_ref[128:, :], preferred_element_type=jnp.float32)
    y += jnp.dot(hlo_ref[...], wr_ref[:128, :], preferred_element_type=jnp.float32)
    y += jnp.dot(hhi_ref[...], wr_ref[128:, :], preferred_element_type=jnp.float32)
    y = y + b_ref[...]
    if relu:
        y = jnp.maximum(y, 0.0)
    lo_ref[...] = y[:, :128]
    hi_ref[...] = y[:, 128:]


def _combine(mean2, hlo, hhi, wl, bl, wr, relu):
    n = hlo.shape[0]
    grid = (n // _RB,)
    return pl.pallas_call(
        functools.partial(_comb_body, relu=relu),
        grid=grid,
        in_specs=[
            pl.BlockSpec((2, _RB, 128), lambda i: (0, i, 0)),
            pl.BlockSpec((_RB, 128), lambda i: (i, 0)),
            pl.BlockSpec((_RB, 128), lambda i: (i, 0)),
            pl.BlockSpec((H, H), lambda i: (0, 0)),
            pl.BlockSpec((H, H), lambda i: (0, 0)),
            pl.BlockSpec((1, H), lambda i: (0, 0)),
        ],
        out_specs=[
            pl.BlockSpec((_RB, 128), lambda i: (i, 0)),
            pl.BlockSpec((_RB, 128), lambda i: (i, 0)),
        ],
        out_shape=[
            jax.ShapeDtypeStruct((n, 128), jnp.float32),
            jax.ShapeDtypeStruct((n, 128), jnp.float32),
        ],
    )(mean2, hlo, hhi, wl, wr, bl.reshape(1, H))


def _seg_body(src_hbm, dst_hbm, tlo_hbm, thi_hbm, out_hbm,
              idx_s, idx_d, rows, ones, stage, cntv, acc, cnt, sem):
    c = lax.axis_index("c")
    s = lax.axis_index("s")

    def _zrow(i, _):
        for k in range(8):
            stage[i, pl.ds(16 * k, 16)] = jnp.zeros((16,), jnp.float32)
        ones[i, :] = jnp.ones((16,), jnp.float32)
        return 0
    lax.fori_loop(0, ZROWS, _zrow, 0)

    for r in range(4):
        pltpu.sync_copy(stage, acc.at[pl.ds(s * 640 + r * ZROWS, ZROWS)])
        pltpu.sync_copy(stage.at[:, pl.ds(0, 16)],
                        cnt.at[pl.ds(s * 640 + r * ZROWS, ZROWS)])

    pltpu.sync_copy(src_hbm.at[s], idx_s)
    pltpu.sync_copy(dst_hbm.at[s], idx_d)
    plsc.subcore_barrier()

    def _blk(j, _):
        @pl.when(c == 0)
        def _():
            pltpu.async_copy(tlo_hbm.at[idx_s.at[j]], rows, sem).wait()

        @pl.when(c == 1)
        def _():
            pltpu.async_copy(thi_hbm.at[idx_s.at[j]], rows, sem).wait()

        pltpu.sync_copy(rows, acc.at[idx_d.at[j]], add=True)
        pltpu.sync_copy(ones.at[pl.ds(0, 128)], cnt.at[idx_d.at[j]], add=True)
        return 0
    lax.fori_loop(0, NBLK, _blk, 0)
    plsc.subcore_barrier()

    for chunk in range(5):
        base = s * WROWS + chunk * 125
        pltpu.sync_copy(acc.at[pl.ds(base, 125)], stage.at[pl.ds(0, 125)])
        pltpu.sync_copy(cnt.at[pl.ds(base, 125)], cntv.at[pl.ds(0, 125)])

        def _srow(i, _):
            cv = cntv[i]
            rcp = 1.0 / jnp.maximum(cv, 1.0)
            for k in range(8):
                stage[i, pl.ds(16 * k, 16)] = stage[i, pl.ds(16 * k, 16)] * rcp
            return 0
        lax.fori_loop(0, 125, _srow, 0)
        pltpu.sync_copy(stage.at[pl.ds(0, 125)], out_hbm.at[c, pl.ds(base, 125)])


_seg_call = pl.kernel(
    _seg_body,
    out_type=jax.ShapeDtypeStruct((2, 10000, 128), jnp.float32),
    mesh=_MESH,
    scratch_types=[
        pltpu.VMEM((NBLK, 128), jnp.int32),
        pltpu.VMEM((NBLK, 128), jnp.int32),
        pltpu.VMEM((128, 128), jnp.float32),
        pltpu.VMEM((ZROWS, 16), jnp.float32),
        pltpu.VMEM((ZROWS, 128), jnp.float32),
        pltpu.VMEM((ZROWS, 16), jnp.float32),
        pltpu.VMEM_SHARED((ACC_ROWS, 128), jnp.float32),
        pltpu.VMEM_SHARED((ACC_ROWS, 16), jnp.float32),
        pltpu.SemaphoreType.DMA,
    ],
)


def _cls_body(i0_hbm, i1_hbm, mlo_hbm, mhi_hbm, dlo_hbm, dhi_hbm, out_hbm,
              i0, i1, bmlo, bmhi, bdlo, bdhi, ob, sem):
    c = lax.axis_index("c")
    s = lax.axis_index("s")
    w = s * NC + c

    pltpu.sync_copy(i0_hbm.at[w], i0)
    pltpu.sync_copy(i1_hbm.at[w], i1)

    for j in range(4):
        pltpu.async_copy(mlo_hbm.at[i0.at[j]], bmlo, sem).wait()
        pltpu.async_copy(mhi_hbm.at[i0.at[j]], bmhi, sem).wait()
        pltpu.async_copy(dlo_hbm.at[i1.at[j]], bdlo, sem).wait()
        pltpu.async_copy(dhi_hbm.at[i1.at[j]], bdhi, sem).wait()

        def _grp(g, _):
            rows16 = lax.iota(jnp.int32, 16) + g * 16

            def _col(cc, acc):
                cols = jnp.full((16,), cc, jnp.int32)
                a = plsc.load_gather(bmlo, [rows16, cols])
                b = plsc.load_gather(bdlo, [rows16, cols])
                acc = acc + a * b
                a = plsc.load_gather(bmhi, [rows16, cols])
                b = plsc.load_gather(bdhi, [rows16, cols])
                return acc + a * b

            acc = lax.fori_loop(0, 128, _col, jnp.zeros((16,), jnp.float32))
            ob[j * 8 + g] = acc
            return 0
        lax.fori_loop(0, 8, _grp, 0)

    pltpu.sync_copy(ob, out_hbm.at[w])


_cls_call = pl.kernel(
    _cls_body,
    out_type=jax.ShapeDtypeStruct((32, 32, 16), jnp.float32),
    mesh=_MESH,
    scratch_types=[
        pltpu.VMEM((4, 128), jnp.int32),
        pltpu.VMEM((4, 128), jnp.int32),
        pltpu.VMEM((128, 128), jnp.float32),
        pltpu.VMEM((128, 128), jnp.float32),
        pltpu.VMEM((128, 128), jnp.float32),
        pltpu.VMEM((128, 128), jnp.float32),
        pltpu.VMEM((32, 16), jnp.float32),
        pltpu.SemaphoreType.DMA,
    ],
)


def _prep_edges(ei):
    src = jnp.pad(ei[0], (0, EPAD - E), constant_values=0)
    dst = jnp.pad(ei[1], (0, EPAD - E), constant_values=PAD_DST)
    return src.reshape(NS, NBLK, 128), dst.reshape(NS, NBLK, 128)


def kernel(x_model, x_dataset, edge_index_m2d, edge_index_d2m, edge_label_index,
           W_in_m, b_in_m, W_in_d, b_in_d,
           conv1_m2d_Wl, conv1_m2d_bl, conv1_m2d_Wr,
           conv1_d2m_Wl, conv1_d2m_bl, conv1_d2m_Wr,
           conv2_m2d_Wl, conv2_m2d_bl, conv2_m2d_Wr,
           conv2_d2m_Wl, conv2_d2m_bl, conv2_d2m_Wr):
    src_m2d, dst_m2d = _prep_edges(edge_index_m2d)
    src_d2m, dst_d2m = _prep_edges(edge_index_d2m)

    hm_lo, hm_hi = _proj(x_model, W_in_m, b_in_m)
    hd_lo, hd_hi = _proj(x_dataset, W_in_d, b_in_d)

    mean1_d = _seg_call(src_m2d, dst_m2d, hm_lo, hm_hi)
    mean1_m = _seg_call(src_d2m, dst_d2m, hd_lo, hd_hi)

    hd1_lo, hd1_hi = _combine(mean1_d, hd_lo, hd_hi,
                              conv1_m2d_Wl, conv1_m2d_bl, conv1_m2d_Wr, True)
    hm1_lo, hm1_hi = _combine(mean1_m, hm_lo, hm_hi,
                              conv1_d2m_Wl, conv1_d2m_bl, conv1_d2m_Wr, True)

    mean2_d = _seg_call(src_m2d, dst_m2d, hm1_lo, hm1_hi)
    mean2_m = _seg_call(src_d2m, dst_d2m, hd1_lo, hd1_hi)

    hd2_lo, hd2_hi = _combine(mean2_d, hd1_lo, hd1_hi,
                              conv2_m2d_Wl, conv2_m2d_bl, conv2_m2d_Wr, False)
    hm2_lo, hm2_hi = _combine(mean2_m, hm1_lo, hm1_hi,
                              conv2_d2m_Wl, conv2_d2m_bl, conv2_d2m_Wr, False)

    i0 = edge_label_index[0].reshape(32, 4, 128)
    i1 = edge_label_index[1].reshape(32, 4, 128)
    out = _cls_call(i0, i1, hm2_lo, hm2_hi, hd2_lo, hd2_hi)
    return out.reshape(L_EDGES)


# speedup vs baseline: 2.4161x; 2.4161x over previous
import functools

import jax
import jax.numpy as jnp
from jax import lax
from jax.experimental import pallas as pl
from jax.experimental.pallas import tpu as pltpu
from jax.experimental.pallas import tpu_sc as plsc

NM = 10000
ND = 10000
D = 256
H = 256
E = 160000
L_EDGES = 16384

NC = 2
NS = 16
LANES = 16

EPAD = 163840
EPT = EPAD // NS
NBLK = EPT // 128
ACC_ROWS = 10240
PAD_DST = 10016
ZROWS = 160
WROWS = 625

_MESH = plsc.VectorSubcoreMesh(core_axis_name="c", subcore_axis_name="s")
_SC_PARAMS = pltpu.CompilerParams(needs_layout_passes=False)


_RB = 2000


def _proj_body(x_ref, w_ref, b_ref, lo_ref, hi_ref):
    y = jnp.dot(x_ref[...], w_ref[...], preferred_element_type=jnp.float32)
    y = y + b_ref[...]
    lo_ref[...] = y[:, :128]
    hi_ref[...] = y[:, 128:]


def _proj(x, w, b):
    n = x.shape[0]
    grid = (n // _RB,)
    return pl.pallas_call(
        _proj_body,
        grid=grid,
        in_specs=[
            pl.BlockSpec((_RB, D), lambda i: (i, 0)),
            pl.BlockSpec((D, H), lambda i: (0, 0)),
            pl.BlockSpec((1, H), lambda i: (0, 0)),
        ],
        out_specs=[
            pl.BlockSpec((_RB, 128), lambda i: (i, 0)),
            pl.BlockSpec((_RB, 128), lambda i: (i, 0)),
        ],
        out_shape=[
            jax.ShapeDtypeStruct((n, 128), jnp.float32),
            jax.ShapeDtypeStruct((n, 128), jnp.float32),
        ],
    )(x, w, b.reshape(1, H))


def _comb_body(m_ref, rcp_ref, hlo_ref, hhi_ref, wl_ref, wr_ref, b_ref,
               lo_ref, hi_ref, *, relu):
    rcp = rcp_ref[0][:, 0:1]
    mlo = m_ref[0] * rcp
    mhi = m_ref[1] * rcp
    y = jnp.dot(mlo, wl_ref[:128, :], preferred_element_type=jnp.float32)
    y += jnp.dot(mhi, wl_ref[128:, :], preferred_element_type=jnp.float32)
    y += jnp.dot(hlo_ref[...], wr_ref[:128, :], preferred_element_type=jnp.float32)
    y += jnp.dot(hhi_ref[...], wr_ref[128:, :], preferred_element_type=jnp.float32)
    y = y + b_ref[...]
    if relu:
        y = jnp.maximum(y, 0.0)
    lo_ref[...] = y[:, :128]
    hi_ref[...] = y[:, 128:]


def _combine(agg2, rcps, e_idx, hlo, hhi, wl, bl, wr, relu):
    n = hlo.shape[0]
    grid = (n // _RB,)
    return pl.pallas_call(
        functools.partial(_comb_body, relu=relu),
        grid=grid,
        in_specs=[
            pl.BlockSpec((2, _RB, 128), lambda i: (0, i, 0)),
            pl.BlockSpec((1, _RB, 128), lambda i, e=e_idx: (e, i, 0)),
            pl.BlockSpec((_RB, 128), lambda i: (i, 0)),
            pl.BlockSpec((_RB, 128), lambda i: (i, 0)),
            pl.BlockSpec((H, H), lambda i: (0, 0)),
            pl.BlockSpec((H, H), lambda i: (0, 0)),
            pl.BlockSpec((1, H), lambda i: (0, 0)),
        ],
        out_specs=[
            pl.BlockSpec((_RB, 128), lambda i: (i, 0)),
            pl.BlockSpec((_RB, 128), lambda i: (i, 0)),
        ],
        out_shape=[
            jax.ShapeDtypeStruct((n, 128), jnp.float32),
            jax.ShapeDtypeStruct((n, 128), jnp.float32),
        ],
    )(agg2, rcps, hlo, hhi, wl, wr, bl.reshape(1, H))


def _seg_body(src_hbm, dst_hbm, tlo_hbm, thi_hbm, out_hbm,
              idx_s, idx_d, rows, acc, sem):
    c = lax.axis_index("c")
    s = lax.axis_index("s")

    def _zrow(i, _):
        for k in range(8):
            rows[i, pl.ds(16 * k, 16)] = jnp.zeros((16,), jnp.float32)
        return 0
    lax.fori_loop(0, 128, _zrow, 0)
    for r in range(5):
        pltpu.sync_copy(rows, acc.at[pl.ds(s * 640 + r * 128, 128)])

    pltpu.sync_copy(src_hbm.at[s], idx_s)
    pltpu.sync_copy(dst_hbm.at[s], idx_d)
    plsc.subcore_barrier()

    def _blk(j, _):
        @pl.when(c == 0)
        def _():
            pltpu.async_copy(tlo_hbm.at[idx_s.at[j]], rows, sem).wait()

        @pl.when(c == 1)
        def _():
            pltpu.async_copy(thi_hbm.at[idx_s.at[j]], rows, sem).wait()

        pltpu.sync_copy(rows, acc.at[idx_d.at[j]], add=True)
        return 0
    lax.fori_loop(0, NBLK, _blk, 0)
    plsc.subcore_barrier()

    for r in range(5):
        base = s * 640 + r * 128
        pltpu.sync_copy(acc.at[pl.ds(base, 128)], rows)
        pltpu.sync_copy(rows, out_hbm.at[c, pl.ds(base, 128)])


_seg_call = pl.kernel(
    _seg_body,
    out_type=jax.ShapeDtypeStruct((2, ACC_ROWS, 128), jnp.float32),
    mesh=_MESH,
    compiler_params=_SC_PARAMS,
    scratch_types=[
        pltpu.VMEM((NBLK, 128), jnp.int32),
        pltpu.VMEM((NBLK, 128), jnp.int32),
        pltpu.VMEM((128, 128), jnp.float32),
        pltpu.VMEM_SHARED((ACC_ROWS, 128), jnp.float32),
        pltpu.SemaphoreType.DMA,
    ],
)


def _deg_body(dst_m2d_hbm, dst_d2m_hbm, out_hbm, idx_d, ones, stage, cnt, sem):
    c = lax.axis_index("c")
    s = lax.axis_index("s")

    def _fill(i, _):
        for k in range(8):
            ones[i, pl.ds(16 * k, 16)] = jnp.ones((16,), jnp.float32)
        return 0
    lax.fori_loop(0, 128, _fill, 0)

    def _zs(i, _):
        for k in range(8):
            stage[i, pl.ds(16 * k, 16)] = jnp.zeros((16,), jnp.float32)
        return 0
    lax.fori_loop(0, 80, _zs, 0)
    for r in range(8):
        pltpu.sync_copy(stage, cnt.at[pl.ds(s * 640 + r * 80, 80)])

    @pl.when(c == 0)
    def _():
        pltpu.sync_copy(dst_m2d_hbm.at[s], idx_d)

    @pl.when(c == 1)
    def _():
        pltpu.sync_copy(dst_d2m_hbm.at[s], idx_d)
    plsc.subcore_barrier()

    def _blk(j, _):
        pltpu.sync_copy(ones, cnt.at[idx_d.at[j]], add=True)
        return 0
    lax.fori_loop(0, NBLK, _blk, 0)
    plsc.subcore_barrier()

    for r in range(8):
        base = s * 640 + r * 80
        pltpu.sync_copy(cnt.at[pl.ds(base, 80)], stage)

        def _srow(i, _):
            for k in range(8):
                v = stage[i, pl.ds(16 * k, 16)]
                stage[i, pl.ds(16 * k, 16)] = 1.0 / jnp.maximum(v, 1.0)
            return 0
        lax.fori_loop(0, 80, _srow, 0)
        pltpu.sync_copy(stage, out_hbm.at[c, pl.ds(base, 80)])


_deg_call = pl.kernel(
    _deg_body,
    out_type=jax.ShapeDtypeStruct((2, ACC_ROWS, 128), jnp.float32),
    mesh=_MESH,
    compiler_params=_SC_PARAMS,
    scratch_types=[
        pltpu.VMEM((NBLK, 128), jnp.int32),
        pltpu.VMEM((128, 128), jnp.float32),
        pltpu.VMEM((80, 128), jnp.float32),
        pltpu.VMEM_SHARED((ACC_ROWS, 128), jnp.float32),
        pltpu.SemaphoreType.DMA,
    ],
)


def _cls_body(i0_hbm, i1_hbm, mlo_hbm, mhi_hbm, dlo_hbm, dhi_hbm, out_hbm,
              i0, i1, bmlo, bmhi, bdlo, bdhi, ob, sem):
    c = lax.axis_index("c")
    s = lax.axis_index("s")
    w = s * NC + c

    pltpu.sync_copy(i0_hbm.at[w], i0)
    pltpu.sync_copy(i1_hbm.at[w], i1)

    for j in range(4):
        pltpu.async_copy(mlo_hbm.at[i0.at[j]], bmlo, sem).wait()
        pltpu.async_copy(mhi_hbm.at[i0.at[j]], bmhi, sem).wait()
        pltpu.async_copy(dlo_hbm.at[i1.at[j]], bdlo, sem).wait()
        pltpu.async_copy(dhi_hbm.at[i1.at[j]], bdhi, sem).wait()

        def _grp(g, _):
            rows16 = lax.iota(jnp.int32, 16) + g * 16

            def _col(cc, acc):
                cols = jnp.full((16,), cc, jnp.int32)
                a = plsc.load_gather(bmlo, [rows16, cols])
                b = plsc.load_gather(bdlo, [rows16, cols])
                acc = acc + a * b
                a = plsc.load_gather(bmhi, [rows16, cols])
                b = plsc.load_gather(bdhi, [rows16, cols])
                return acc + a * b

            acc = lax.fori_loop(0, 128, _col, jnp.zeros((16,), jnp.float32))
            ob[j * 8 + g] = acc
            return 0
        lax.fori_loop(0, 8, _grp, 0)

    pltpu.sync_copy(ob, out_hbm.at[w])


_cls_call = pl.kernel(
    _cls_body,
    out_type=jax.ShapeDtypeStruct((32, 32, 16), jnp.float32),
    mesh=_MESH,
    compiler_params=_SC_PARAMS,
    scratch_types=[
        pltpu.VMEM((4, 128), jnp.int32),
        pltpu.VMEM((4, 128), jnp.int32),
        pltpu.VMEM((128, 128), jnp.float32),
        pltpu.VMEM((128, 128), jnp.float32),
        pltpu.VMEM((128, 128), jnp.float32),
        pltpu.VMEM((128, 128), jnp.float32),
        pltpu.VMEM((32, 16), jnp.float32),
        pltpu.SemaphoreType.DMA,
    ],
)


def _prep_edges(ei):
    src = jnp.pad(ei[0], (0, EPAD - E), constant_values=0)
    dst = jnp.pad(ei[1], (0, EPAD - E), constant_values=PAD_DST)
    return src.reshape(NS, NBLK, 128), dst.reshape(NS, NBLK, 128)


def kernel(x_model, x_dataset, edge_index_m2d, edge_index_d2m, edge_label_index,
           W_in_m, b_in_m, W_in_d, b_in_d,
           conv1_m2d_Wl, conv1_m2d_bl, conv1_m2d_Wr,
           conv1_d2m_Wl, conv1_d2m_bl, conv1_d2m_Wr,
           conv2_m2d_Wl, conv2_m2d_bl, conv2_m2d_Wr,
           conv2_d2m_Wl, conv2_d2m_bl, conv2_d2m_Wr):
    src_m2d, dst_m2d = _prep_edges(edge_index_m2d)
    src_d2m, dst_d2m = _prep_edges(edge_index_d2m)

    rcps = _deg_call(dst_m2d, dst_d2m)

    hm_lo, hm_hi = _proj(x_model, W_in_m, b_in_m)
    hd_lo, hd_hi = _proj(x_dataset, W_in_d, b_in_d)

    agg1_d = _seg_call(src_m2d, dst_m2d, hm_lo, hm_hi)
    agg1_m = _seg_call(src_d2m, dst_d2m, hd_lo, hd_hi)

    hd1_lo, hd1_hi = _combine(agg1_d, rcps, 0, hd_lo, hd_hi,
                              conv1_m2d_Wl, conv1_m2d_bl, conv1_m2d_Wr, True)
    hm1_lo, hm1_hi = _combine(agg1_m, rcps, 1, hm_lo, hm_hi,
                              conv1_d2m_Wl, conv1_d2m_bl, conv1_d2m_Wr, True)

    agg2_d = _seg_call(src_m2d, dst_m2d, hm1_lo, hm1_hi)
    agg2_m = _seg_call(src_d2m, dst_d2m, hd1_lo, hd1_hi)

    hd2_lo, hd2_hi = _combine(agg2_d, rcps, 0, hd1_lo, hd1_hi,
                              conv2_m2d_Wl, conv2_m2d_bl, conv2_m2d_Wr, False)
    hm2_lo, hm2_hi = _combine(agg2_m, rcps, 1, hm1_lo, hm1_hi,
                              conv2_d2m_Wl, conv2_d2m_bl, conv2_d2m_Wr, False)

    i0 = edge_label_index[0].reshape(32, 4, 128)
    i1 = edge_label_index[1].reshape(32, 4, 128)
    out = _cls_call(i0, i1, hm2_lo, hm2_hi, hd2_lo, hd2_hi)
    return out.reshape(L_EDGES)


# speedup vs baseline: 2.5727x; 1.0648x over previous
import functools

import jax
import jax.numpy as jnp
from jax import lax
from jax.experimental import pallas as pl
from jax.experimental.pallas import tpu as pltpu
from jax.experimental.pallas import tpu_sc as plsc

NM = 10000
ND = 10000
D = 256
H = 256
E = 160000
L_EDGES = 16384

NC = 2
NS = 16

EPAD = 163840
EPT = EPAD // NS
BLK = 80
NBLK = EPT // BLK
ACC_ROWS = 10240
PAD_DST = 10016

_MESH = plsc.VectorSubcoreMesh(core_axis_name="c", subcore_axis_name="s")
_SC_PARAMS = pltpu.CompilerParams(needs_layout_passes=False)


_RB = 2000


def _proj_body(x_ref, w_ref, b_ref, o_ref):
    o_ref[...] = (jnp.dot(x_ref[...], w_ref[...],
                          preferred_element_type=jnp.float32) + b_ref[...])


def _proj(x, w, b):
    n = x.shape[0]
    nb = n // _RB
    return pl.pallas_call(
        _proj_body,
        grid=(nb, 2),
        in_specs=[
            pl.BlockSpec((_RB, D), lambda i, h: (i, 0)),
            pl.BlockSpec((D, 128), lambda i, h: (0, h)),
            pl.BlockSpec((1, 128), lambda i, h: (0, h)),
        ],
        out_specs=pl.BlockSpec((_RB, 128), lambda i, h, nb=nb: (h * nb + i, 0)),
        out_shape=jax.ShapeDtypeStruct((2 * n, 128), jnp.float32),
    )(x, w, b.reshape(1, H))


def _comb_body(m_ref, rcp_ref, hlo_ref, hhi_ref, wl_ref, wr_ref, b_ref, o_ref,
               *, relu):
    rcp = rcp_ref[0][:, 0:1]
    y = jnp.dot(m_ref[0] * rcp, wl_ref[:128, :],
                preferred_element_type=jnp.float32)
    y += jnp.dot(m_ref[1] * rcp, wl_ref[128:, :],
                 preferred_element_type=jnp.float32)
    y += jnp.dot(hlo_ref[...], wr_ref[:128, :],
                 preferred_element_type=jnp.float32)
    y += jnp.dot(hhi_ref[...], wr_ref[128:, :],
                 preferred_element_type=jnp.float32)
    y = y + b_ref[...]
    if relu:
        y = jnp.maximum(y, 0.0)
    o_ref[...] = y


def _combine(agg2, rcps, e_idx, h2n, wl, bl, wr, relu):
    n = h2n.shape[0] // 2
    nb = n // _RB
    return pl.pallas_call(
        functools.partial(_comb_body, relu=relu),
        grid=(nb, 2),
        in_specs=[
            pl.BlockSpec((2, _RB, 128), lambda i, h: (0, i, 0)),
            pl.BlockSpec((1, _RB, 128), lambda i, h, e=e_idx: (e, i, 0)),
            pl.BlockSpec((_RB, 128), lambda i, h: (i, 0)),
            pl.BlockSpec((_RB, 128), lambda i, h, nb=nb: (nb + i, 0)),
            pl.BlockSpec((H, 128), lambda i, h: (0, h)),
            pl.BlockSpec((H, 128), lambda i, h: (0, h)),
            pl.BlockSpec((1, 128), lambda i, h: (0, h)),
        ],
        out_specs=pl.BlockSpec((_RB, 128), lambda i, h, nb=nb: (h * nb + i, 0)),
        out_shape=jax.ShapeDtypeStruct((2 * n, 128), jnp.float32),
    )(agg2, rcps, h2n, h2n, wl, wr, bl.reshape(1, H))


def _seg_body(src_hbm, dst_hbm, tbl_hbm, out_hbm,
              idx_s, idx_d, rows0, rows1, acc, gs0, gs1, ss0, ss1):
    c = lax.axis_index("c")
    s = lax.axis_index("s")

    def _zrow(i, _):
        for k in range(8):
            rows0[i, pl.ds(16 * k, 16)] = jnp.zeros((16,), jnp.float32)
        return 0
    lax.fori_loop(0, BLK, _zrow, 0)
    for r in range(8):
        pltpu.sync_copy(rows0, acc.at[pl.ds(s * 640 + r * BLK, BLK)])

    pltpu.sync_copy(src_hbm.at[s], idx_s)
    pltpu.sync_copy(dst_hbm.at[s, pl.ds(0, 32)], idx_d)
    half = tbl_hbm.shape[0] // 2
    off = jnp.full((16,), c * half, jnp.int32)

    def _adj(i, _):
        idx_s[pl.ds(16 * i, 16)] = idx_s[pl.ds(16 * i, 16)] + off
        return 0
    lax.fori_loop(0, EPT // 16, _adj, 0)
    plsc.subcore_barrier()

    def _sblk(j):
        return idx_s.at[pl.ds(j * BLK, BLK)]

    pltpu.async_copy(tbl_hbm.at[_sblk(0)], rows0, gs0)
    pltpu.async_copy(tbl_hbm.at[_sblk(1)], rows1, gs1)

    def _pair(jj, _):
        j = 2 * jj

        @pl.when(jnp.logical_and(jj > 0, jj % 16 == 0))
        def _():
            pltpu.sync_copy(dst_hbm.at[s, pl.ds(pl.multiple_of(j, 32), 32)],
                            idx_d)

        jl = j & 31
        pltpu.make_async_copy(tbl_hbm.at[_sblk(j)], rows0, gs0).wait()
        pltpu.async_copy(rows0, acc.at[idx_d.at[jl]], ss0, add=True)
        pltpu.make_async_copy(tbl_hbm.at[_sblk(j + 1)], rows1, gs1).wait()
        pltpu.async_copy(rows1, acc.at[idx_d.at[jl + 1]], ss1, add=True)

        @pl.when(jj < NBLK // 2 - 1)
        def _():
            pltpu.make_async_copy(rows0, acc.at[idx_d.at[jl]], ss0).wait()
            pltpu.async_copy(tbl_hbm.at[_sblk(j + 2)], rows0, gs0)
            pltpu.make_async_copy(rows1, acc.at[idx_d.at[jl + 1]], ss1).wait()
            pltpu.async_copy(tbl_hbm.at[_sblk(j + 3)], rows1, gs1)
        return 0
    lax.fori_loop(0, NBLK // 2, _pair, 0)
    pltpu.make_async_copy(rows0, acc.at[idx_d.at[30]], ss0).wait()
    pltpu.make_async_copy(rows1, acc.at[idx_d.at[31]], ss1).wait()
    plsc.subcore_barrier()

    for r in range(8):
        base = s * 640 + r * BLK
        pltpu.sync_copy(acc.at[pl.ds(base, BLK)], rows0)
        pltpu.sync_copy(rows0, out_hbm.at[c, pl.ds(base, BLK)])


_seg_call = pl.kernel(
    _seg_body,
    out_type=jax.ShapeDtypeStruct((2, ACC_ROWS, 128), jnp.float32),
    mesh=_MESH,
    compiler_params=_SC_PARAMS,
    scratch_types=[
        pltpu.VMEM((EPT,), jnp.int32),
        pltpu.VMEM((32, BLK), jnp.int32),
        pltpu.VMEM((BLK, 128), jnp.float32),
        pltpu.VMEM((BLK, 128), jnp.float32),
        pltpu.VMEM_SHARED((ACC_ROWS, 128), jnp.float32),
        pltpu.SemaphoreType.DMA,
        pltpu.SemaphoreType.DMA,
        pltpu.SemaphoreType.DMA,
        pltpu.SemaphoreType.DMA,
    ],
)


def _deg_body(dst_m2d_hbm, dst_d2m_hbm, out_hbm, idx_d, ones, stage, cnt, sem):
    c = lax.axis_index("c")
    s = lax.axis_index("s")

    def _fill(i, _):
        for k in range(8):
            ones[i, pl.ds(16 * k, 16)] = jnp.ones((16,), jnp.float32)
            stage[i, pl.ds(16 * k, 16)] = jnp.zeros((16,), jnp.float32)
        return 0
    lax.fori_loop(0, BLK, _fill, 0)
    for r in range(8):
        pltpu.sync_copy(stage, cnt.at[pl.ds(s * 640 + r * BLK, BLK)])

    @pl.when(c == 0)
    def _():
        pltpu.sync_copy(dst_m2d_hbm.at[s], idx_d)

    @pl.when(c == 1)
    def _():
        pltpu.sync_copy(dst_d2m_hbm.at[s], idx_d)
    plsc.subcore_barrier()

    def _blk(j, _):
        pltpu.sync_copy(ones, cnt.at[idx_d.at[j]], add=True)
        return 0
    lax.fori_loop(0, NBLK, _blk, 0)
    plsc.subcore_barrier()

    for r in range(8):
        base = s * 640 + r * BLK
        pltpu.sync_copy(cnt.at[pl.ds(base, BLK)], stage)

        def _srow(i, _):
            for k in range(8):
                v = stage[i, pl.ds(16 * k, 16)]
                stage[i, pl.ds(16 * k, 16)] = 1.0 / jnp.maximum(v, 1.0)
            return 0
        lax.fori_loop(0, BLK, _srow, 0)
        pltpu.sync_copy(stage, out_hbm.at[c, pl.ds(base, BLK)])


_deg_call = pl.kernel(
    _deg_body,
    out_type=jax.ShapeDtypeStruct((2, ACC_ROWS, 128), jnp.float32),
    mesh=_MESH,
    compiler_params=_SC_PARAMS,
    scratch_types=[
        pltpu.VMEM((NBLK, BLK), jnp.int32),
        pltpu.VMEM((BLK, 128), jnp.float32),
        pltpu.VMEM((BLK, 128), jnp.float32),
        pltpu.VMEM_SHARED((ACC_ROWS, 128), jnp.float32),
        pltpu.SemaphoreType.DMA,
    ],
)


_CB = 4


def _cls_body(i0_hbm, i1_hbm, m_hbm, d_hbm, out_hbm,
              i0, i0h, i1, i1h, bmlo, bmhi, bdlo, bdhi, ob, sem):
    c = lax.axis_index("c")
    s = lax.axis_index("s")
    w = s * NC + c

    pltpu.sync_copy(i0_hbm.at[w], i0)
    pltpu.sync_copy(i1_hbm.at[w], i1)
    offm = jnp.full((16,), m_hbm.shape[0] // 2, jnp.int32)
    offd = jnp.full((16,), d_hbm.shape[0] // 2, jnp.int32)

    def _mkhi(i, _):
        for k in range(8):
            i0h[i, pl.ds(16 * k, 16)] = i0[i, pl.ds(16 * k, 16)] + offm
            i1h[i, pl.ds(16 * k, 16)] = i1[i, pl.ds(16 * k, 16)] + offd
        return 0
    lax.fori_loop(0, _CB, _mkhi, 0)

    for j in range(_CB):
        d1 = pltpu.async_copy(m_hbm.at[i0.at[j]], bmlo, sem)
        d2 = pltpu.async_copy(m_hbm.at[i0h.at[j]], bmhi, sem)
        d3 = pltpu.async_copy(d_hbm.at[i1.at[j]], bdlo, sem)
        d4 = pltpu.async_copy(d_hbm.at[i1h.at[j]], bdhi, sem)
        d1.wait()
        d2.wait()
        d3.wait()
        d4.wait()

        def _grp(g, _):
            rows16 = lax.iota(jnp.int32, 16) + g * 16

            def _col(cc, acc):
                cols = jnp.full((16,), cc, jnp.int32)
                a = plsc.load_gather(bmlo, [rows16, cols])
                b = plsc.load_gather(bdlo, [rows16, cols])
                acc = acc + a * b
                a = plsc.load_gather(bmhi, [rows16, cols])
                b = plsc.load_gather(bdhi, [rows16, cols])
                return acc + a * b

            acc = lax.fori_loop(0, 128, _col, jnp.zeros((16,), jnp.float32))
            ob[j * 8 + g] = acc
            return 0
        lax.fori_loop(0, 8, _grp, 0)

    pltpu.sync_copy(ob, out_hbm.at[w])


_cls_call = pl.kernel(
    _cls_body,
    out_type=jax.ShapeDtypeStruct((32, 32, 16), jnp.float32),
    mesh=_MESH,
    compiler_params=_SC_PARAMS,
    scratch_types=[
        pltpu.VMEM((_CB, 128), jnp.int32),
        pltpu.VMEM((_CB, 128), jnp.int32),
        pltpu.VMEM((_CB, 128), jnp.int32),
        pltpu.VMEM((_CB, 128), jnp.int32),
        pltpu.VMEM((128, 128), jnp.float32),
        pltpu.VMEM((128, 128), jnp.float32),
        pltpu.VMEM((128, 128), jnp.float32),
        pltpu.VMEM((128, 128), jnp.float32),
        pltpu.VMEM((32, 16), jnp.float32),
        pltpu.SemaphoreType.DMA,
    ],
)


def _prep_edges(ei):
    src = jnp.pad(ei[0], (0, EPAD - E), constant_values=0)
    dst = jnp.pad(ei[1], (0, EPAD - E), constant_values=PAD_DST)
    return src.reshape(NS, EPT), dst.reshape(NS, NBLK, BLK)


def kernel(x_model, x_dataset, edge_index_m2d, edge_index_d2m, edge_label_index,
           W_in_m, b_in_m, W_in_d, b_in_d,
           conv1_m2d_Wl, conv1_m2d_bl, conv1_m2d_Wr,
           conv1_d2m_Wl, conv1_d2m_bl, conv1_d2m_Wr,
           conv2_m2d_Wl, conv2_m2d_bl, conv2_m2d_Wr,
           conv2_d2m_Wl, conv2_d2m_bl, conv2_d2m_Wr):
    src_m2d, dst_m2d = _prep_edges(edge_index_m2d)
    src_d2m, dst_d2m = _prep_edges(edge_index_d2m)

    rcps = _deg_call(dst_m2d, dst_d2m)

    hm = _proj(x_model, W_in_m, b_in_m)
    hd = _proj(x_dataset, W_in_d, b_in_d)

    agg1_d = _seg_call(src_m2d, dst_m2d, hm)
    agg1_m = _seg_call(src_d2m, dst_d2m, hd)

    hd1 = _combine(agg1_d, rcps, 0, hd,
                   conv1_m2d_Wl, conv1_m2d_bl, conv1_m2d_Wr, True)
    hm1 = _combine(agg1_m, rcps, 1, hm,
                   conv1_d2m_Wl, conv1_d2m_bl, conv1_d2m_Wr, True)

    agg2_d = _seg_call(src_m2d, dst_m2d, hm1)
    agg2_m = _seg_call(src_d2m, dst_d2m, hd1)

    hd2 = _combine(agg2_d, rcps, 0, hd1,
                   conv2_m2d_Wl, conv2_m2d_bl, conv2_m2d_Wr, False)
    hm2 = _combine(agg2_m, rcps, 1, hm1,
                   conv2_d2m_Wl, conv2_d2m_bl, conv2_d2m_Wr, False)

    i0 = edge_label_index[0].reshape(32, _CB, 128)
    i1 = edge_label_index[1].reshape(32, _CB, 128)
    out = _cls_call(i0, i1, hm2, hd2)
    return out.reshape(L_EDGES)


# speedup vs baseline: 2.6165x; 1.0170x over previous
import functools

import jax
import jax.numpy as jnp
from jax import lax
from jax.experimental import pallas as pl
from jax.experimental.pallas import tpu as pltpu
from jax.experimental.pallas import tpu_sc as plsc

NM = 10000
ND = 10000
D = 256
H = 256
E = 160000
L_EDGES = 16384

NC = 2
NS = 16

EPAD = 163840
EPT = EPAD // NS
BLK = 64
NBLK = EPT // BLK
NQ = NBLK // 4
ACC_ROWS = 10112
RPT = ACC_ROWS // NS
PAD_DST = 10016
_CHUNKS = [(r * 64, 64) for r in range(9)] + [(576, 56)]

_MESH = plsc.VectorSubcoreMesh(core_axis_name="c", subcore_axis_name="s")
_SC_PARAMS = pltpu.CompilerParams(needs_layout_passes=False)


_RB = 2000


def _proj_body(x_ref, w_ref, b_ref, o_ref):
    o_ref[...] = (jnp.dot(x_ref[...], w_ref[...],
                          preferred_element_type=jnp.float32) + b_ref[...])


def _proj(x, w, b):
    n = x.shape[0]
    nb = n // _RB
    return pl.pallas_call(
        _proj_body,
        grid=(nb, 2),
        in_specs=[
            pl.BlockSpec((_RB, D), lambda i, h: (i, 0)),
            pl.BlockSpec((D, 128), lambda i, h: (0, h)),
            pl.BlockSpec((1, 128), lambda i, h: (0, h)),
        ],
        out_specs=pl.BlockSpec((_RB, 128), lambda i, h, nb=nb: (h * nb + i, 0)),
        out_shape=jax.ShapeDtypeStruct((2 * n, 128), jnp.float32),
    )(x, w, b.reshape(1, H))


def _comb_body(m_ref, rcp_ref, hlo_ref, hhi_ref, wl_ref, wr_ref, b_ref, o_ref,
               *, relu):
    rcp = rcp_ref[0][:, 0:1]
    y = jnp.dot(m_ref[0] * rcp, wl_ref[:128, :],
                preferred_element_type=jnp.float32)
    y += jnp.dot(m_ref[1] * rcp, wl_ref[128:, :],
                 preferred_element_type=jnp.float32)
    y += jnp.dot(hlo_ref[...], wr_ref[:128, :],
                 preferred_element_type=jnp.float32)
    y += jnp.dot(hhi_ref[...], wr_ref[128:, :],
                 preferred_element_type=jnp.float32)
    y = y + b_ref[...]
    if relu:
        y = jnp.maximum(y, 0.0)
    o_ref[...] = y


def _combine(agg2, rcps, e_idx, h2n, wl, bl, wr, relu):
    n = h2n.shape[0] // 2
    nb = n // _RB
    return pl.pallas_call(
        functools.partial(_comb_body, relu=relu),
        grid=(nb, 2),
        in_specs=[
            pl.BlockSpec((2, _RB, 128), lambda i, h: (0, i, 0)),
            pl.BlockSpec((1, _RB, 128), lambda i, h, e=e_idx: (e, i, 0)),
            pl.BlockSpec((_RB, 128), lambda i, h: (i, 0)),
            pl.BlockSpec((_RB, 128), lambda i, h, nb=nb: (nb + i, 0)),
            pl.BlockSpec((H, 128), lambda i, h: (0, h)),
            pl.BlockSpec((H, 128), lambda i, h: (0, h)),
            pl.BlockSpec((1, 128), lambda i, h: (0, h)),
        ],
        out_specs=pl.BlockSpec((_RB, 128), lambda i, h, nb=nb: (h * nb + i, 0)),
        out_shape=jax.ShapeDtypeStruct((2 * n, 128), jnp.float32),
    )(agg2, rcps, h2n, h2n, wl, wr, bl.reshape(1, H))


def _seg_body(src_hbm, dst_hbm, tbl_hbm, out_hbm,
              idx_s, idx_d, rows0, rows1, rows2, rows3, acc,
              gs0, gs1, gs2, gs3, ss0, ss1, ss2, ss3):
    c = lax.axis_index("c")
    s = lax.axis_index("s")
    rows = (rows0, rows1, rows2, rows3)
    gs = (gs0, gs1, gs2, gs3)
    ss = (ss0, ss1, ss2, ss3)

    def _zrow(i, _):
        for k in range(8):
            rows0[i, pl.ds(16 * k, 16)] = jnp.zeros((16,), jnp.float32)
        return 0
    lax.fori_loop(0, BLK, _zrow, 0)
    for base, size in _CHUNKS:
        pltpu.sync_copy(rows0.at[pl.ds(0, size)],
                        acc.at[pl.ds(s * RPT + base, size)])

    pltpu.sync_copy(src_hbm.at[s], idx_s)
    pltpu.sync_copy(dst_hbm.at[s, pl.ds(0, 32)], idx_d)
    half = tbl_hbm.shape[0] // 2
    off = jnp.full((16,), c * half, jnp.int32)

    def _adj(i, _):
        idx_s[pl.ds(16 * i, 16)] = idx_s[pl.ds(16 * i, 16)] + off
        return 0
    lax.fori_loop(0, EPT // 16, _adj, 0)
    plsc.subcore_barrier()

    def _sblk(j):
        return idx_s.at[pl.ds(j * BLK, BLK)]

    for b in range(4):
        pltpu.async_copy(tbl_hbm.at[_sblk(b)], rows[b], gs[b])

    def _quad(ii, _):
        j = 4 * ii

        @pl.when(jnp.logical_and(ii > 0, ii % 8 == 0))
        def _():
            pltpu.sync_copy(dst_hbm.at[s, pl.ds(pl.multiple_of(j, 32), 32)],
                            idx_d)

        jl = j & 31
        for b in range(4):
            pltpu.make_async_copy(tbl_hbm.at[_sblk(j + b)],
                                  rows[b], gs[b]).wait()
            pltpu.async_copy(rows[b], acc.at[idx_d.at[jl + b]], ss[b],
                             add=True)

        @pl.when(ii < NQ - 1)
        def _():
            for b in range(4):
                pltpu.make_async_copy(rows[b], acc.at[idx_d.at[jl + b]],
                                      ss[b]).wait()
                pltpu.async_copy(tbl_hbm.at[_sblk(j + 4 + b)], rows[b], gs[b])
        return 0
    lax.fori_loop(0, NQ, _quad, 0)
    for b in range(4):
        pltpu.make_async_copy(rows[b], acc.at[idx_d.at[28 + b]], ss[b]).wait()
    plsc.subcore_barrier()

    for base, size in _CHUNKS:
        pltpu.sync_copy(acc.at[pl.ds(s * RPT + base, size)],
                        rows0.at[pl.ds(0, size)])
        pltpu.sync_copy(rows0.at[pl.ds(0, size)],
                        out_hbm.at[c, pl.ds(s * RPT + base, size)])


_seg_call = pl.kernel(
    _seg_body,
    out_type=jax.ShapeDtypeStruct((2, ACC_ROWS, 128), jnp.float32),
    mesh=_MESH,
    compiler_params=_SC_PARAMS,
    scratch_types=[
        pltpu.VMEM((EPT,), jnp.int32),
        pltpu.VMEM((32, BLK), jnp.int32),
        pltpu.VMEM((BLK, 128), jnp.float32),
        pltpu.VMEM((BLK, 128), jnp.float32),
        pltpu.VMEM((BLK, 128), jnp.float32),
        pltpu.VMEM((BLK, 128), jnp.float32),
        pltpu.VMEM_SHARED((ACC_ROWS, 128), jnp.float32),
        pltpu.SemaphoreType.DMA,
        pltpu.SemaphoreType.DMA,
        pltpu.SemaphoreType.DMA,
        pltpu.SemaphoreType.DMA,
        pltpu.SemaphoreType.DMA,
        pltpu.SemaphoreType.DMA,
        pltpu.SemaphoreType.DMA,
        pltpu.SemaphoreType.DMA,
    ],
)


def _deg_body(dst_m2d_hbm, dst_d2m_hbm, out_hbm, idx_d, ones, stage, cnt, sem):
    c = lax.axis_index("c")
    s = lax.axis_index("s")

    def _fill(i, _):
        for k in range(8):
            ones[i, pl.ds(16 * k, 16)] = jnp.ones((16,), jnp.float32)
            stage[i, pl.ds(16 * k, 16)] = jnp.zeros((16,), jnp.float32)
        return 0
    lax.fori_loop(0, BLK, _fill, 0)
    for base, size in _CHUNKS:
        pltpu.sync_copy(stage.at[pl.ds(0, size)],
                        cnt.at[pl.ds(s * RPT + base, size)])

    @pl.when(c == 0)
    def _():
        pltpu.sync_copy(dst_m2d_hbm.at[s], idx_d)

    @pl.when(c == 1)
    def _():
        pltpu.sync_copy(dst_d2m_hbm.at[s], idx_d)
    plsc.subcore_barrier()

    def _blk(j, _):
        pltpu.sync_copy(ones, cnt.at[idx_d.at[j]], add=True)
        return 0
    lax.fori_loop(0, NBLK, _blk, 0)
    plsc.subcore_barrier()

    for base, size in _CHUNKS:
        gbase = s * RPT + base
        pltpu.sync_copy(cnt.at[pl.ds(gbase, size)], stage.at[pl.ds(0, size)])

        def _srow(i, _):
            for k in range(8):
                v = stage[i, pl.ds(16 * k, 16)]
                stage[i, pl.ds(16 * k, 16)] = 1.0 / jnp.maximum(v, 1.0)
            return 0
        lax.fori_loop(0, size, _srow, 0)
        pltpu.sync_copy(stage.at[pl.ds(0, size)],
                        out_hbm.at[c, pl.ds(gbase, size)])


_deg_call = pl.kernel(
    _deg_body,
    out_type=jax.ShapeDtypeStruct((2, ACC_ROWS, 128), jnp.float32),
    mesh=_MESH,
    compiler_params=_SC_PARAMS,
    scratch_types=[
        pltpu.VMEM((NBLK, BLK), jnp.int32),
        pltpu.VMEM((BLK, 128), jnp.float32),
        pltpu.VMEM((BLK, 128), jnp.float32),
        pltpu.VMEM_SHARED((ACC_ROWS, 128), jnp.float32),
        pltpu.SemaphoreType.DMA,
    ],
)


_CB = 4


def _cls_body(i0_hbm, i1_hbm, m_hbm, d_hbm, out_hbm,
              i0, i0h, i1, i1h, bmlo, bmhi, bdlo, bdhi, ob, sem):
    c = lax.axis_index("c")
    s = lax.axis_index("s")
    w = s * NC + c

    pltpu.sync_copy(i0_hbm.at[w], i0)
    pltpu.sync_copy(i1_hbm.at[w], i1)
    offm = jnp.full((16,), m_hbm.shape[0] // 2, jnp.int32)
    offd = jnp.full((16,), d_hbm.shape[0] // 2, jnp.int32)

    def _mkhi(i, _):
        for k in range(8):
            i0h[i, pl.ds(16 * k, 16)] = i0[i, pl.ds(16 * k, 16)] + offm
            i1h[i, pl.ds(16 * k, 16)] = i1[i, pl.ds(16 * k, 16)] + offd
        return 0
    lax.fori_loop(0, _CB, _mkhi, 0)

    for j in range(_CB):
        d1 = pltpu.async_copy(m_hbm.at[i0.at[j]], bmlo, sem)
        d2 = pltpu.async_copy(m_hbm.at[i0h.at[j]], bmhi, sem)
        d3 = pltpu.async_copy(d_hbm.at[i1.at[j]], bdlo, sem)
        d4 = pltpu.async_copy(d_hbm.at[i1h.at[j]], bdhi, sem)
        d1.wait()
        d2.wait()
        d3.wait()
        d4.wait()

        def _grp(g, _):
            rows16 = lax.iota(jnp.int32, 16) + g * 16

            def _col(cc, acc):
                cols = jnp.full((16,), cc, jnp.int32)
                a = plsc.load_gather(bmlo, [rows16, cols])
                b = plsc.load_gather(bdlo, [rows16, cols])
                acc = acc + a * b
                a = plsc.load_gather(bmhi, [rows16, cols])
                b = plsc.load_gather(bdhi, [rows16, cols])
                return acc + a * b

            acc = lax.fori_loop(0, 128, _col, jnp.zeros((16,), jnp.float32))
            ob[j * 8 + g] = acc
            return 0
        lax.fori_loop(0, 8, _grp, 0)

    pltpu.sync_copy(ob, out_hbm.at[w])


_cls_call = pl.kernel(
    _cls_body,
    out_type=jax.ShapeDtypeStruct((32, 32, 16), jnp.float32),
    mesh=_MESH,
    compiler_params=_SC_PARAMS,
    scratch_types=[
        pltpu.VMEM((_CB, 128), jnp.int32),
        pltpu.VMEM((_CB, 128), jnp.int32),
        pltpu.VMEM((_CB, 128), jnp.int32),
        pltpu.VMEM((_CB, 128), jnp.int32),
        pltpu.VMEM((128, 128), jnp.float32),
        pltpu.VMEM((128, 128), jnp.float32),
        pltpu.VMEM((128, 128), jnp.float32),
        pltpu.VMEM((128, 128), jnp.float32),
        pltpu.VMEM((32, 16), jnp.float32),
        pltpu.SemaphoreType.DMA,
    ],
)


def _prep_edges(ei):
    src = jnp.pad(ei[0], (0, EPAD - E), constant_values=0)
    dst = jnp.pad(ei[1], (0, EPAD - E), constant_values=PAD_DST)
    return src.reshape(NS, EPT), dst.reshape(NS, NBLK, BLK)


def kernel(x_model, x_dataset, edge_index_m2d, edge_index_d2m, edge_label_index,
           W_in_m, b_in_m, W_in_d, b_in_d,
           conv1_m2d_Wl, conv1_m2d_bl, conv1_m2d_Wr,
           conv1_d2m_Wl, conv1_d2m_bl, conv1_d2m_Wr,
           conv2_m2d_Wl, conv2_m2d_bl, conv2_m2d_Wr,
           conv2_d2m_Wl, conv2_d2m_bl, conv2_d2m_Wr):
    src_m2d, dst_m2d = _prep_edges(edge_index_m2d)
    src_d2m, dst_d2m = _prep_edges(edge_index_d2m)

    rcps = _deg_call(dst_m2d, dst_d2m)

    hm = _proj(x_model, W_in_m, b_in_m)
    hd = _proj(x_dataset, W_in_d, b_in_d)

    agg1_d = _seg_call(src_m2d, dst_m2d, hm)
    agg1_m = _seg_call(src_d2m, dst_d2m, hd)

    hd1 = _combine(agg1_d, rcps, 0, hd,
                   conv1_m2d_Wl, conv1_m2d_bl, conv1_m2d_Wr, True)
    hm1 = _combine(agg1_m, rcps, 1, hm,
                   conv1_d2m_Wl, conv1_d2m_bl, conv1_d2m_Wr, True)

    agg2_d = _seg_call(src_m2d, dst_m2d, hm1)
    agg2_m = _seg_call(src_d2m, dst_d2m, hd1)

    hd2 = _combine(agg2_d, rcps, 0, hd1,
                   conv2_m2d_Wl, conv2_m2d_bl, conv2_m2d_Wr, False)
    hm2 = _combine(agg2_m, rcps, 1, hm1,
                   conv2_d2m_Wl, conv2_d2m_bl, conv2_d2m_Wr, False)

    i0 = edge_label_index[0].reshape(32, _CB, 128)
    i1 = edge_label_index[1].reshape(32, _CB, 128)
    out = _cls_call(i0, i1, hm2, hd2)
    return out.reshape(L_EDGES)


# speedup vs baseline: 2.8998x; 1.1082x over previous
import functools

import jax
import jax.numpy as jnp
from jax import lax
from jax.experimental import pallas as pl
from jax.experimental.pallas import tpu as pltpu
from jax.experimental.pallas import tpu_sc as plsc

NM = 10000
ND = 10000
D = 256
H = 256
E = 160000
L_EDGES = 16384

NC = 2
NS = 16

EPAD = 163840
EPT = EPAD // NS
BLK = 64
NBLK = EPT // BLK
NQ = NBLK // 4
ACC_ROWS = 10112
RPT = ACC_ROWS // NS
PAD_DST = 10016
_CHUNKS = [(r * 64, 64) for r in range(9)] + [(576, 56)]

_MESH = plsc.VectorSubcoreMesh(core_axis_name="c", subcore_axis_name="s")
_SC_PARAMS = pltpu.CompilerParams(needs_layout_passes=False)


_RB = 2000


def _proj_body(x_ref, w_ref, b_ref, o_ref):
    o_ref[...] = (jnp.dot(x_ref[...], w_ref[...],
                          preferred_element_type=jnp.float32) + b_ref[...])


def _proj(x, w, b):
    n = x.shape[0]
    nb = n // _RB
    return pl.pallas_call(
        _proj_body,
        grid=(nb, 2),
        in_specs=[
            pl.BlockSpec((_RB, D), lambda i, h: (i, 0)),
            pl.BlockSpec((D, 128), lambda i, h: (0, h)),
            pl.BlockSpec((1, 128), lambda i, h: (0, h)),
        ],
        out_specs=pl.BlockSpec((_RB, 128), lambda i, h, nb=nb: (h * nb + i, 0)),
        out_shape=jax.ShapeDtypeStruct((2 * n, 128), jnp.float32),
    )(x, w, b.reshape(1, H))


def _comb_body(m_ref, rcp_ref, hlo_ref, hhi_ref, wl_ref, wr_ref, b_ref, o_ref,
               *, relu):
    rcp = rcp_ref[0][:, 0:1]
    y = jnp.dot(m_ref[0] * rcp, wl_ref[:128, :],
                preferred_element_type=jnp.float32)
    y += jnp.dot(m_ref[1] * rcp, wl_ref[128:, :],
                 preferred_element_type=jnp.float32)
    y += jnp.dot(hlo_ref[...], wr_ref[:128, :],
                 preferred_element_type=jnp.float32)
    y += jnp.dot(hhi_ref[...], wr_ref[128:, :],
                 preferred_element_type=jnp.float32)
    y = y + b_ref[...]
    if relu:
        y = jnp.maximum(y, 0.0)
    o_ref[...] = y


def _combine(agg2, rcps, e_idx, h2n, wl, bl, wr, relu):
    n = h2n.shape[0] // 2
    nb = n // _RB
    return pl.pallas_call(
        functools.partial(_comb_body, relu=relu),
        grid=(nb, 2),
        in_specs=[
            pl.BlockSpec((2, _RB, 128), lambda i, h: (0, i, 0)),
            pl.BlockSpec((1, _RB, 128), lambda i, h, e=e_idx: (e, i, 0)),
            pl.BlockSpec((_RB, 128), lambda i, h: (i, 0)),
            pl.BlockSpec((_RB, 128), lambda i, h, nb=nb: (nb + i, 0)),
            pl.BlockSpec((H, 128), lambda i, h: (0, h)),
            pl.BlockSpec((H, 128), lambda i, h: (0, h)),
            pl.BlockSpec((1, 128), lambda i, h: (0, h)),
        ],
        out_specs=pl.BlockSpec((_RB, 128), lambda i, h, nb=nb: (h * nb + i, 0)),
        out_shape=jax.ShapeDtypeStruct((2 * n, 128), jnp.float32),
    )(agg2, rcps, h2n, h2n, wl, wr, bl.reshape(1, H))


def _seg_body(src_hbm, dst_hbm, tbl_hbm, out_hbm,
              idx_s, idx_d, rows0, rows1, rows2, rows3, acc,
              gs0, gs1, gs2, gs3, ss0, ss1, ss2, ss3):
    c = lax.axis_index("c")
    s = lax.axis_index("s")
    rows = (rows0, rows1, rows2, rows3)
    gs = (gs0, gs1, gs2, gs3)
    ss = (ss0, ss1, ss2, ss3)

    def _zrow(i, _):
        for k in range(8):
            rows0[i, pl.ds(16 * k, 16)] = jnp.zeros((16,), jnp.float32)
        return 0
    lax.fori_loop(0, BLK, _zrow, 0)
    for base, size in _CHUNKS:
        pltpu.sync_copy(rows0.at[pl.ds(0, size)],
                        acc.at[pl.ds(s * RPT + base, size)])

    pltpu.sync_copy(src_hbm.at[s], idx_s)
    pltpu.sync_copy(dst_hbm.at[s, pl.ds(0, 32)], idx_d)
    half = tbl_hbm.shape[0] // 2
    off = jnp.full((16,), c * half, jnp.int32)

    def _adj(i, _):
        idx_s[pl.ds(16 * i, 16)] = idx_s[pl.ds(16 * i, 16)] + off
        return 0
    lax.fori_loop(0, EPT // 16, _adj, 0)
    plsc.subcore_barrier()

    def _sblk(j):
        return idx_s.at[pl.ds(j * BLK, BLK)]

    for b in range(4):
        pltpu.async_copy(tbl_hbm.at[_sblk(b)], rows[b], gs[b])

    def _quad(ii, _):
        j = 4 * ii

        @pl.when(jnp.logical_and(ii > 0, ii % 8 == 0))
        def _():
            pltpu.sync_copy(dst_hbm.at[s, pl.ds(pl.multiple_of(j, 32), 32)],
                            idx_d)

        jl = j & 31
        for b in range(4):
            pltpu.make_async_copy(tbl_hbm.at[_sblk(j + b)],
                                  rows[b], gs[b]).wait()
            pltpu.async_copy(rows[b], acc.at[idx_d.at[jl + b]], ss[b],
                             add=True)

        @pl.when(ii < NQ - 1)
        def _():
            for b in range(4):
                pltpu.make_async_copy(rows[b], acc.at[idx_d.at[jl + b]],
                                      ss[b]).wait()
                pltpu.async_copy(tbl_hbm.at[_sblk(j + 4 + b)], rows[b], gs[b])
        return 0
    lax.fori_loop(0, NQ, _quad, 0)
    for b in range(4):
        pltpu.make_async_copy(rows[b], acc.at[idx_d.at[28 + b]], ss[b]).wait()
    plsc.subcore_barrier()

    for base, size in _CHUNKS:
        pltpu.sync_copy(acc.at[pl.ds(s * RPT + base, size)],
                        rows0.at[pl.ds(0, size)])
        pltpu.sync_copy(rows0.at[pl.ds(0, size)],
                        out_hbm.at[c, pl.ds(s * RPT + base, size)])


_seg_call = pl.kernel(
    _seg_body,
    out_type=jax.ShapeDtypeStruct((2, ACC_ROWS, 128), jnp.float32),
    mesh=_MESH,
    compiler_params=_SC_PARAMS,
    scratch_types=[
        pltpu.VMEM((EPT,), jnp.int32),
        pltpu.VMEM((32, BLK), jnp.int32),
        pltpu.VMEM((BLK, 128), jnp.float32),
        pltpu.VMEM((BLK, 128), jnp.float32),
        pltpu.VMEM((BLK, 128), jnp.float32),
        pltpu.VMEM((BLK, 128), jnp.float32),
        pltpu.VMEM_SHARED((ACC_ROWS, 128), jnp.float32),
        pltpu.SemaphoreType.DMA,
        pltpu.SemaphoreType.DMA,
        pltpu.SemaphoreType.DMA,
        pltpu.SemaphoreType.DMA,
        pltpu.SemaphoreType.DMA,
        pltpu.SemaphoreType.DMA,
        pltpu.SemaphoreType.DMA,
        pltpu.SemaphoreType.DMA,
    ],
)


def _deg_body(dst_m2d_hbm, dst_d2m_hbm, out_hbm, idx_d, ones, stage, cnt, sem):
    c = lax.axis_index("c")
    s = lax.axis_index("s")

    def _fill(i, _):
        for k in range(8):
            ones[i, pl.ds(16 * k, 16)] = jnp.ones((16,), jnp.float32)
            stage[i, pl.ds(16 * k, 16)] = jnp.zeros((16,), jnp.float32)
        return 0
    lax.fori_loop(0, BLK, _fill, 0)
    for base, size in _CHUNKS:
        pltpu.sync_copy(stage.at[pl.ds(0, size)],
                        cnt.at[pl.ds(s * RPT + base, size)])

    @pl.when(c == 0)
    def _():
        pltpu.sync_copy(dst_m2d_hbm.at[s], idx_d)

    @pl.when(c == 1)
    def _():
        pltpu.sync_copy(dst_d2m_hbm.at[s], idx_d)
    plsc.subcore_barrier()

    def _blk(j, _):
        pltpu.sync_copy(ones, cnt.at[idx_d.at[j]], add=True)
        return 0
    lax.fori_loop(0, NBLK, _blk, 0)
    plsc.subcore_barrier()

    for base, size in _CHUNKS:
        gbase = s * RPT + base
        pltpu.sync_copy(cnt.at[pl.ds(gbase, size)], stage.at[pl.ds(0, size)])

        def _srow(i, _):
            for k in range(8):
                v = stage[i, pl.ds(16 * k, 16)]
                stage[i, pl.ds(16 * k, 16)] = 1.0 / jnp.maximum(v, 1.0)
            return 0
        lax.fori_loop(0, size, _srow, 0)
        pltpu.sync_copy(stage.at[pl.ds(0, size)],
                        out_hbm.at[c, pl.ds(gbase, size)])


_deg_call = pl.kernel(
    _deg_body,
    out_type=jax.ShapeDtypeStruct((2, ACC_ROWS, 128), jnp.float32),
    mesh=_MESH,
    compiler_params=_SC_PARAMS,
    scratch_types=[
        pltpu.VMEM((NBLK, BLK), jnp.int32),
        pltpu.VMEM((BLK, 128), jnp.float32),
        pltpu.VMEM((BLK, 128), jnp.float32),
        pltpu.VMEM_SHARED((ACC_ROWS, 128), jnp.float32),
        pltpu.SemaphoreType.DMA,
    ],
)


_CB = 8
_CBS = 64


def _cls_body(i0_hbm, i1_hbm, m_hbm, d_hbm, out_hbm,
              i0, i0h, i1, i1h,
              bmlo0, bmhi0, bdlo0, bdhi0, bmlo1, bmhi1, bdlo1, bdhi1,
              tbuf, ob, sem0, sem1):
    c = lax.axis_index("c")
    s = lax.axis_index("s")
    w = s * NC + c
    bufs = ((bmlo0, bmhi0, bdlo0, bdhi0), (bmlo1, bmhi1, bdlo1, bdhi1))
    sems = (sem0, sem1)

    pltpu.sync_copy(i0_hbm.at[w], i0)
    pltpu.sync_copy(i1_hbm.at[w], i1)
    offm = jnp.full((16,), m_hbm.shape[0] // 2, jnp.int32)
    offd = jnp.full((16,), d_hbm.shape[0] // 2, jnp.int32)

    def _mkhi(i, _):
        for k in range(4):
            i0h[i, pl.ds(16 * k, 16)] = i0[i, pl.ds(16 * k, 16)] + offm
            i1h[i, pl.ds(16 * k, 16)] = i1[i, pl.ds(16 * k, 16)] + offd
        return 0
    lax.fori_loop(0, _CB, _mkhi, 0)

    def _fire(j, p):
        ml, mh, dl, dh = bufs[p]
        return (pltpu.async_copy(m_hbm.at[i0.at[j]], ml, sems[p]),
                pltpu.async_copy(m_hbm.at[i0h.at[j]], mh, sems[p]),
                pltpu.async_copy(d_hbm.at[i1.at[j]], dl, sems[p]),
                pltpu.async_copy(d_hbm.at[i1h.at[j]], dh, sems[p]))

    descs = _fire(0, 0)
    for j in range(_CB):
        if j + 1 < _CB:
            nxt = _fire(j + 1, (j + 1) & 1)
        for dd in descs:
            dd.wait()
        ml, mh, dl, dh = bufs[j & 1]

        def _grp(g, _):
            base = g * 16

            def _row(r, _):
                i = base + r
                p = None
                for k in range(8):
                    sl = pl.ds(16 * k, 16)
                    t = ml[i, sl] * dl[i, sl] + mh[i, sl] * dh[i, sl]
                    p = t if p is None else p + t
                tbuf[r] = p
                return 0
            lax.fori_loop(0, 16, _row, 0)
            rows16 = lax.iota(jnp.int32, 16)
            acc = jnp.zeros((16,), jnp.float32)
            for t in range(16):
                acc = acc + plsc.load_gather(
                    tbuf, [rows16, jnp.full((16,), t, jnp.int32)])
            ob[j * 4 + g] = acc
            return 0
        lax.fori_loop(0, 4, _grp, 0)
        if j + 1 < _CB:
            descs = nxt

    pltpu.sync_copy(ob, out_hbm.at[w])


_cls_call = pl.kernel(
    _cls_body,
    out_type=jax.ShapeDtypeStruct((32, 32, 16), jnp.float32),
    mesh=_MESH,
    compiler_params=_SC_PARAMS,
    scratch_types=[
        pltpu.VMEM((_CB, _CBS), jnp.int32),
        pltpu.VMEM((_CB, _CBS), jnp.int32),
        pltpu.VMEM((_CB, _CBS), jnp.int32),
        pltpu.VMEM((_CB, _CBS), jnp.int32),
        pltpu.VMEM((_CBS, 128), jnp.float32),
        pltpu.VMEM((_CBS, 128), jnp.float32),
        pltpu.VMEM((_CBS, 128), jnp.float32),
        pltpu.VMEM((_CBS, 128), jnp.float32),
        pltpu.VMEM((_CBS, 128), jnp.float32),
        pltpu.VMEM((_CBS, 128), jnp.float32),
        pltpu.VMEM((_CBS, 128), jnp.float32),
        pltpu.VMEM((_CBS, 128), jnp.float32),
        pltpu.VMEM((16, 16), jnp.float32),
        pltpu.VMEM((32, 16), jnp.float32),
        pltpu.SemaphoreType.DMA,
        pltpu.SemaphoreType.DMA,
    ],
)


def _prep_edges(ei):
    src = jnp.pad(ei[0], (0, EPAD - E), constant_values=0)
    dst = jnp.pad(ei[1], (0, EPAD - E), constant_values=PAD_DST)
    return src.reshape(NS, EPT), dst.reshape(NS, NBLK, BLK)


def kernel(x_model, x_dataset, edge_index_m2d, edge_index_d2m, edge_label_index,
           W_in_m, b_in_m, W_in_d, b_in_d,
           conv1_m2d_Wl, conv1_m2d_bl, conv1_m2d_Wr,
           conv1_d2m_Wl, conv1_d2m_bl, conv1_d2m_Wr,
           conv2_m2d_Wl, conv2_m2d_bl, conv2_m2d_Wr,
           conv2_d2m_Wl, conv2_d2m_bl, conv2_d2m_Wr):
    src_m2d, dst_m2d = _prep_edges(edge_index_m2d)
    src_d2m, dst_d2m = _prep_edges(edge_index_d2m)

    rcps = _deg_call(dst_m2d, dst_d2m)

    hm = _proj(x_model, W_in_m, b_in_m)
    hd = _proj(x_dataset, W_in_d, b_in_d)

    agg1_d = _seg_call(src_m2d, dst_m2d, hm)
    agg1_m = _seg_call(src_d2m, dst_d2m, hd)

    hd1 = _combine(agg1_d, rcps, 0, hd,
                   conv1_m2d_Wl, conv1_m2d_bl, conv1_m2d_Wr, True)
    hm1 = _combine(agg1_m, rcps, 1, hm,
                   conv1_d2m_Wl, conv1_d2m_bl, conv1_d2m_Wr, True)

    agg2_d = _seg_call(src_m2d, dst_m2d, hm1)
    agg2_m = _seg_call(src_d2m, dst_d2m, hd1)

    hd2 = _combine(agg2_d, rcps, 0, hd1,
                   conv2_m2d_Wl, conv2_m2d_bl, conv2_m2d_Wr, False)
    hm2 = _combine(agg2_m, rcps, 1, hm1,
                   conv2_d2m_Wl, conv2_d2m_bl, conv2_d2m_Wr, False)

    i0 = edge_label_index[0].reshape(32, _CB, _CBS)
    i1 = edge_label_index[1].reshape(32, _CB, _CBS)
    out = _cls_call(i0, i1, hm2, hd2)
    return out.reshape(L_EDGES)
